# Initial kernel scaffold; baseline (speedup 1.0000x reference)
import jax
import jax.numpy as jnp
from jax.experimental import pallas as pl


def kernel(boxes, scores, classes):
    raise NotImplementedError("write your pallas kernel here")


# speedup vs baseline: 87.3127x; 87.3127x over previous
import functools

import jax
import jax.numpy as jnp
from jax import lax
from jax.experimental import pallas as pl
from jax.experimental.pallas import tpu as pltpu
from jax.experimental.pallas import tpu_sc as plsc

N = 20000
NUM_RES = 16
NP = 20480
CHW = 1280
CAP = 4096
CAPR = CAP // 128
CAPP = CAP + 16
TOPK = 200
OUTP = 256
IOU_THR = 0.45
GRPS = CAP // 16
NGRP = N // 16


def _iota16():
    return lax.iota(jnp.int32, 16)


def _nms_body(orderp1, x1c, y1c, x2c, y2c, scc, clsc,
              ox1, oy1, ox2, oy2, osc, ocl,
              och_v, cg_v, cb_v, pos_v, cls_v, orig_v,
              x1_v, y1_v, x2_v, y2_v, ar_v, keep_v, sup_v,
              cnt16_v, cntall_v, tpos_v, tsup_v, keepg_v,
              r2d_v, o200_v, ox1_v, oy1_v, ox2_v, oy2_v, osc_v,
              ocl_v, ocli_v,
              sh_c, sh_pos, sh_sup, sh_cnt):
    w = lax.axis_index("s")
    iota = _iota16()

    pltpu.sync_copy(orderp1.at[pl.ds(w * CHW, CHW)], och_v)
    for j in range(CHW // 128):
        sl = pl.ds(j * 128, 128)
        pltpu.sync_copy(clsc.at[och_v.at[sl]], cg_v.at[sl])
    pltpu.sync_copy(cg_v, sh_c.at[pl.ds(w * CHW, CHW)])
    plsc.subcore_barrier()

    def init_b(g, _):
        off = g * 16
        keep_v[pl.ds(off, 16)] = jnp.ones((16,), jnp.int32)
        cls_v[pl.ds(off, 16)] = jnp.full((16,), -1, jnp.int32)
        pos_v[pl.ds(off, 16)] = jnp.zeros((16,), jnp.int32)
        return 0
    lax.fori_loop(0, GRPS, init_b, 0)

    def filt_chunk(kc, cnt):
        pltpu.sync_copy(sh_c.at[pl.ds(kc * CHW, CHW)], cb_v)

        def filt_g(g, cnt):
            cvec = cb_v[pl.ds(g * 16, 16)]
            pvec = kc * CHW + g * 16 + iota
            m = ((cvec & (NUM_RES - 1)) == w) & (pvec < N)
            mi = m.astype(jnp.int32)
            q = cnt + plsc.cumsum(mi) - mi
            m = m & (q < CAP)
            qc = jnp.where(m, q, 0)
            plsc.store_scatter(pos_v, [qc], pvec, mask=m)
            plsc.store_scatter(cls_v, [qc], cvec, mask=m)
            return cnt + jnp.sum(mi)
        return lax.fori_loop(0, CHW // 16, filt_g, cnt)
    cnt = lax.fori_loop(0, NP // CHW, filt_chunk, jnp.int32(0))
    cnt = jnp.minimum(cnt, CAP)

    nch = (cnt + 127) // 128

    def gath(j, _):
        sl = pl.ds(j * 128, 128)
        pltpu.sync_copy(orderp1.at[pos_v.at[sl]], orig_v.at[sl])
        pltpu.sync_copy(x1c.at[orig_v.at[sl]], x1_v.at[sl])
        pltpu.sync_copy(y1c.at[orig_v.at[sl]], y1_v.at[sl])
        pltpu.sync_copy(x2c.at[orig_v.at[sl]], x2_v.at[sl])
        pltpu.sync_copy(y2c.at[orig_v.at[sl]], y2_v.at[sl])
        return 0
    lax.fori_loop(0, nch, gath, 0)

    def area_b(g, _):
        sl = pl.ds(g * 16, 16)
        wd = jnp.maximum(x2_v[sl] - x1_v[sl], 0.0)
        ht = jnp.maximum(y2_v[sl] - y1_v[sl], 0.0)
        ar_v[sl] = wd * ht
        return 0
    lax.fori_loop(0, GRPS, area_b, 0)

    ngrp = (cnt + 15) // 16

    def nms_i(i, _):
        sli = pl.ds(i, 16)
        ki = keep_v[sli][0]

        @pl.when(ki != 0)
        def _():
            xi = jnp.full((16,), x1_v[sli][0], jnp.float32)
            yi = jnp.full((16,), y1_v[sli][0], jnp.float32)
            mXi = jnp.full((16,), x2_v[sli][0], jnp.float32)
            mYi = jnp.full((16,), y2_v[sli][0], jnp.float32)
            ai = jnp.full((16,), ar_v[sli][0], jnp.float32)
            ci = jnp.full((16,), cls_v[sli][0], jnp.int32)
            iv = jnp.full((16,), i, jnp.int32)

            def nms_g(g, _):
                off = g * 16
                sl = pl.ds(off, 16)
                jv = off + iota
                xx1 = jnp.maximum(xi, x1_v[sl])
                yy1 = jnp.maximum(yi, y1_v[sl])
                xx2 = jnp.minimum(mXi, x2_v[sl])
                yy2 = jnp.minimum(mYi, y2_v[sl])
                inter = jnp.maximum(xx2 - xx1, 0.0) * jnp.maximum(yy2 - yy1, 0.0)
                iou = inter / (ai + ar_v[sl] - inter + 1e-9)
                sup = (iou > IOU_THR) & (cls_v[sl] == ci) & (jv > iv)
                keep_v[sl] = jnp.where(sup, 0, keep_v[sl])
                return 0
            lax.fori_loop(i // 16, ngrp, nms_g, 0)
        return 0
    lax.fori_loop(0, cnt, nms_i, 0)

    def sup_b(g, _):
        sl = pl.ds(g * 16, 16)
        jv = g * 16 + iota
        sup_v[sl] = ((keep_v[sl] == 0) & (jv < cnt)).astype(jnp.int32)
        return 0
    lax.fori_loop(0, GRPS, sup_b, 0)
    pltpu.sync_copy(pos_v, sh_pos.at[w])
    pltpu.sync_copy(sup_v, sh_sup.at[w])
    cnt16_v[...] = jnp.full((16,), cnt, jnp.int32)
    pltpu.sync_copy(cnt16_v, sh_cnt.at[pl.ds(w * 16, 16)])
    plsc.subcore_barrier()

    @pl.when(w == 0)
    def _():
        pltpu.sync_copy(sh_cnt, cntall_v)

        def merge_t(t, _):
            pltpu.sync_copy(sh_pos.at[t], tpos_v)
            pltpu.sync_copy(sh_sup.at[t], tsup_v)
            ct = cntall_v[pl.ds(t * 16, 16)][0]

            def merge_g(g, _):
                off = g * 16
                jv = off + iota
                pvec = tpos_v[pl.ds(off, 16)]
                s16 = tsup_v[pl.ds(off, 16)]
                m = jv < ct
                pc = jnp.where(m, pvec, 0)
                plsc.store_scatter(keepg_v, [pc], s16, mask=m)
                return 0
            lax.fori_loop(0, (ct + 15) // 16, merge_g, 0)
            return 0
        lax.fori_loop(0, NUM_RES, merge_t, 0)

        def kt_b(g, acc):
            sup = keepg_v[pl.ds(g * 16, 16)]
            return acc + jnp.sum(1 - sup)
        ktot = lax.fori_loop(0, NGRP, kt_b, jnp.int32(0))

        for k in range(OUTP // 16):
            r2d_v[pl.ds(k * 16, 16)] = jnp.zeros((16,), jnp.int32)

        def part_b(g, c):
            nk, ns = c
            sup = keepg_v[pl.ds(g * 16, 16)]
            kii = (sup == 0).astype(jnp.int32)
            sii = 1 - kii
            ck = plsc.cumsum(kii) - kii
            cs = plsc.cumsum(sii) - sii
            pvec = g * 16 + iota
            po = jnp.where(kii == 1, nk + ck, ktot + ns + cs)
            m = po < TOPK
            pc = jnp.where(m, po, 0)
            plsc.store_scatter(r2d_v, [pc], pvec, mask=m)
            return nk + jnp.sum(kii), ns + jnp.sum(sii)
        lax.fori_loop(0, NGRP, part_b, (jnp.int32(0), jnp.int32(0)))

        for j in range(2):
            sl = pl.ds(j * 128, 128)
            pltpu.sync_copy(orderp1.at[r2d_v.at[sl]], o200_v.at[sl])
            pltpu.sync_copy(x1c.at[o200_v.at[sl]], ox1_v.at[sl])
            pltpu.sync_copy(y1c.at[o200_v.at[sl]], oy1_v.at[sl])
            pltpu.sync_copy(x2c.at[o200_v.at[sl]], ox2_v.at[sl])
            pltpu.sync_copy(y2c.at[o200_v.at[sl]], oy2_v.at[sl])
            pltpu.sync_copy(scc.at[o200_v.at[sl]], osc_v.at[sl])
            pltpu.sync_copy(clsc.at[o200_v.at[sl]], ocli_v.at[sl])
        for g in range(OUTP // 16):
            sl = pl.ds(g * 16, 16)
            rv = g * 16 + iota
            osc_v[sl] = jnp.where(rv < ktot, osc_v[sl], -1.0)
            ocl_v[sl] = ocli_v[sl].astype(jnp.float32)
        pltpu.sync_copy(ox1_v, ox1)
        pltpu.sync_copy(oy1_v, oy1)
        pltpu.sync_copy(ox2_v, ox2)
        pltpu.sync_copy(oy2_v, oy2)
        pltpu.sync_copy(osc_v, osc)
        pltpu.sync_copy(ocl_v, ocl)


@jax.jit
def kernel(boxes, scores, classes):
    order = jnp.argsort(-scores).astype(jnp.int32)
    orderp1 = jnp.concatenate(
        [order, jnp.zeros((NP - N,), jnp.int32)])
    x1c = boxes[:, 0]
    y1c = boxes[:, 1]
    x2c = boxes[:, 2]
    y2c = boxes[:, 3]
    clsc = classes.astype(jnp.int32)

    f32 = jnp.float32
    i32 = jnp.int32
    out_type = [jax.ShapeDtypeStruct((OUTP,), f32) for _ in range(6)]
    scratch = [
        pltpu.VMEM((CHW,), i32),
        pltpu.VMEM((CHW,), i32),
        pltpu.VMEM((CHW,), i32),
        pltpu.VMEM((CAP,), i32),
        pltpu.VMEM((CAPP,), i32),
        pltpu.VMEM((CAP,), i32),
        pltpu.VMEM((CAPP,), f32),
        pltpu.VMEM((CAPP,), f32),
        pltpu.VMEM((CAPP,), f32),
        pltpu.VMEM((CAPP,), f32),
        pltpu.VMEM((CAPP,), f32),
        pltpu.VMEM((CAPP,), i32),
        pltpu.VMEM((CAP,), i32),
        pltpu.VMEM((16,), i32),
        pltpu.VMEM((16 * NUM_RES,), i32),
        pltpu.VMEM((CAP,), i32),
        pltpu.VMEM((CAP,), i32),
        pltpu.VMEM((NP,), i32),
        pltpu.VMEM((OUTP,), i32),
        pltpu.VMEM((OUTP,), i32),
        pltpu.VMEM((OUTP,), f32),
        pltpu.VMEM((OUTP,), f32),
        pltpu.VMEM((OUTP,), f32),
        pltpu.VMEM((OUTP,), f32),
        pltpu.VMEM((OUTP,), f32),
        pltpu.VMEM((OUTP,), f32),
        pltpu.VMEM((OUTP,), i32),
        pltpu.VMEM_SHARED((NP,), i32),
        pltpu.VMEM_SHARED((NUM_RES, CAP), i32),
        pltpu.VMEM_SHARED((NUM_RES, CAP), i32),
        pltpu.VMEM_SHARED((16 * NUM_RES,), i32),
    ]
    mesh = plsc.VectorSubcoreMesh(
        core_axis_name="c", subcore_axis_name="s", num_cores=1)
    run = pl.kernel(
        _nms_body, out_type=out_type, mesh=mesh, scratch_types=scratch,
        compiler_params=pltpu.CompilerParams(needs_layout_passes=False))
    ox1, oy1, ox2, oy2, osc, ocl = run(
        orderp1, x1c, y1c, x2c, y2c, scores, clsc)
    ob = jnp.stack([ox1[:TOPK], oy1[:TOPK], ox2[:TOPK], oy2[:TOPK]], axis=1)
    return jnp.concatenate(
        [ob, osc[:TOPK, None], ocl[:TOPK, None]], axis=1)


# speedup vs baseline: 276.1244x; 3.1625x over previous
import jax
import jax.numpy as jnp
from jax import lax
from jax.experimental import pallas as pl
from jax.experimental.pallas import tpu as pltpu
from jax.experimental.pallas import tpu_sc as plsc

N = 20000
NUM_RES = 16
NSEG = 6
CAPC = 512
CAP = NSEG * CAPC
NP = 20480
CHW = 1280
CAPP = CAP + 16
TOPK = 200
OUTP = 256
IOU_THR = 0.45
GRPS = CAP // 16
NGRP = N // 16


def _iota16():
    return lax.iota(jnp.int32, 16)


def _nms_body(orderp1, x1c, y1c, x2c, y2c, scc, clsc,
              ox1, oy1, ox2, oy2, osc, ocl,
              och_v, cg_v, cb_v, pos_v, orig_v,
              x1_v, y1_v, x2_v, y2_v, ar_v, keep_v, sup_v,
              cnt16_v, cntall_v, tpos_v, tsup_v, keepg_v,
              r2d_v, o200_v, ox1_v, oy1_v, ox2_v, oy2_v, osc_v,
              ocl_v, ocli_v,
              sh_c, sh_pos, sh_sup, sh_cnt):
    w = lax.axis_index("s")
    iota = _iota16()

    pltpu.sync_copy(orderp1.at[pl.ds(w * CHW, CHW)], och_v)
    for j in range(CHW // 128):
        sl = pl.ds(j * 128, 128)
        pltpu.sync_copy(clsc.at[och_v.at[sl]], cg_v.at[sl])
    pltpu.sync_copy(cg_v, sh_c.at[pl.ds(w * CHW, CHW)])
    plsc.subcore_barrier()

    def init_b(g, _):
        off = g * 16
        keep_v[pl.ds(off, 16)] = jnp.ones((16,), jnp.int32)
        pos_v[pl.ds(off, 16)] = jnp.zeros((16,), jnp.int32)
        return 0
    lax.fori_loop(0, GRPS, init_b, 0)

    def filt_chunk(kc, cnts):
        pltpu.sync_copy(sh_c.at[pl.ds(kc * CHW, CHW)], cb_v)

        def filt_g(g, cnts):
            cvec = cb_v[pl.ds(g * 16, 16)]
            pvec = kc * CHW + g * 16 + iota
            pin = pvec < N
            new = []
            for k in range(NSEG):
                ck = cnts[k]
                m = (cvec == (w + 16 * k)) & pin
                mi = m.astype(jnp.int32)
                q = k * CAPC + ck + plsc.cumsum(mi) - mi
                m = m & (q < (k + 1) * CAPC)
                qc = jnp.where(m, q, 0)
                plsc.store_scatter(pos_v, [qc], pvec, mask=m)
                new.append(ck + jnp.sum(mi))
            return tuple(new)
        return lax.fori_loop(0, CHW // 16, filt_g, cnts)
    cnts = lax.fori_loop(0, NP // CHW, filt_chunk,
                         tuple(jnp.int32(0) for _ in range(NSEG)))
    cnts = tuple(jnp.minimum(c, CAPC) for c in cnts)
    cntv = jnp.zeros((16,), jnp.int32)
    for k in range(NSEG):
        cntv = jnp.where(iota == k, cnts[k], cntv)
    cnt16_v[...] = cntv

    for k in range(NSEG):
        nch = (cnts[k] + 127) // 128

        def gath(j, _, k=k):
            sl = pl.ds(k * CAPC + j * 128, 128)
            pltpu.sync_copy(orderp1.at[pos_v.at[sl]], orig_v.at[sl])
            pltpu.sync_copy(x1c.at[orig_v.at[sl]], x1_v.at[sl])
            pltpu.sync_copy(y1c.at[orig_v.at[sl]], y1_v.at[sl])
            pltpu.sync_copy(x2c.at[orig_v.at[sl]], x2_v.at[sl])
            pltpu.sync_copy(y2c.at[orig_v.at[sl]], y2_v.at[sl])
            return 0
        lax.fori_loop(0, nch, gath, 0)

        def area_b(g, _, k=k):
            sl = pl.ds(k * CAPC + g * 16, 16)
            wd = jnp.maximum(x2_v[sl] - x1_v[sl], 0.0)
            ht = jnp.maximum(y2_v[sl] - y1_v[sl], 0.0)
            ar_v[sl] = wd * ht
            return 0
        lax.fori_loop(0, (cnts[k] + 15) // 16, area_b, 0)

    for k in range(NSEG):
        base = k * CAPC
        end = base + cnts[k]
        ngrp = (end + 15) // 16

        def nms_i(i, _, end=end, ngrp=ngrp):
            sli = pl.ds(i, 16)
            ki = keep_v[sli][0]

            @pl.when(ki != 0)
            def _():
                xi = jnp.full((16,), x1_v[sli][0], jnp.float32)
                yi = jnp.full((16,), y1_v[sli][0], jnp.float32)
                mXi = jnp.full((16,), x2_v[sli][0], jnp.float32)
                mYi = jnp.full((16,), y2_v[sli][0], jnp.float32)
                ai = jnp.full((16,), ar_v[sli][0], jnp.float32)
                iv = jnp.full((16,), i, jnp.int32)
                ev = jnp.full((16,), end, jnp.int32)

                def nms_g(g, _):
                    off = g * 16
                    sl = pl.ds(off, 16)
                    jv = off + iota
                    xx1 = jnp.maximum(xi, x1_v[sl])
                    yy1 = jnp.maximum(yi, y1_v[sl])
                    xx2 = jnp.minimum(mXi, x2_v[sl])
                    yy2 = jnp.minimum(mYi, y2_v[sl])
                    inter = (jnp.maximum(xx2 - xx1, 0.0)
                             * jnp.maximum(yy2 - yy1, 0.0))
                    iou = inter / (ai + ar_v[sl] - inter + 1e-9)
                    sup = (iou > IOU_THR) & (jv > iv) & (jv < ev)
                    keep_v[sl] = jnp.where(sup, 0, keep_v[sl])
                    return 0
                lax.fori_loop(i // 16, ngrp, nms_g, 0)
            return 0
        lax.fori_loop(base, end, nms_i, 0)

    def sup_b(g, _):
        sl = pl.ds(g * 16, 16)
        jv = g * 16 + iota
        endl = plsc.load_gather(cnt16_v, [jv >> 9])
        sup_v[sl] = ((keep_v[sl] == 0)
                     & ((jv & (CAPC - 1)) < endl)).astype(jnp.int32)
        return 0
    lax.fori_loop(0, GRPS, sup_b, 0)
    pltpu.sync_copy(pos_v, sh_pos.at[w])
    pltpu.sync_copy(sup_v, sh_sup.at[w])
    pltpu.sync_copy(cnt16_v, sh_cnt.at[pl.ds(w * 16, 16)])
    plsc.subcore_barrier()

    @pl.when(w == 0)
    def _():
        pltpu.sync_copy(sh_cnt, cntall_v)

        def merge_t(t, _):
            pltpu.sync_copy(sh_pos.at[t], tpos_v)
            pltpu.sync_copy(sh_sup.at[t], tsup_v)
            cvt = cntall_v[pl.ds(t * 16, 16)]
            for k in range(NSEG):
                ct = cvt[k]

                def merge_g(g, _, k=k, ct=ct):
                    off = k * CAPC + g * 16
                    jv = g * 16 + iota
                    pvec = tpos_v[pl.ds(off, 16)]
                    s16 = tsup_v[pl.ds(off, 16)]
                    m = jv < ct
                    pc = jnp.where(m, pvec, 0)
                    plsc.store_scatter(keepg_v, [pc], s16, mask=m)
                    return 0
                lax.fori_loop(0, (ct + 15) // 16, merge_g, 0)
            return 0
        lax.fori_loop(0, NUM_RES, merge_t, 0)

        def kt_b(g, acc):
            sup = keepg_v[pl.ds(g * 16, 16)]
            return acc + jnp.sum(1 - sup)
        ktot = lax.fori_loop(0, NGRP, kt_b, jnp.int32(0))

        for kk in range(OUTP // 16):
            r2d_v[pl.ds(kk * 16, 16)] = jnp.zeros((16,), jnp.int32)

        def part_b(g, c):
            nk, ns = c
            sup = keepg_v[pl.ds(g * 16, 16)]
            kii = (sup == 0).astype(jnp.int32)
            sii = 1 - kii
            ck = plsc.cumsum(kii) - kii
            cs = plsc.cumsum(sii) - sii
            pvec = g * 16 + iota
            po = jnp.where(kii == 1, nk + ck, ktot + ns + cs)
            m = po < TOPK
            pc = jnp.where(m, po, 0)
            plsc.store_scatter(r2d_v, [pc], pvec, mask=m)
            return nk + jnp.sum(kii), ns + jnp.sum(sii)
        lax.fori_loop(0, NGRP, part_b, (jnp.int32(0), jnp.int32(0)))

        for j in range(2):
            sl = pl.ds(j * 128, 128)
            pltpu.sync_copy(orderp1.at[r2d_v.at[sl]], o200_v.at[sl])
            pltpu.sync_copy(x1c.at[o200_v.at[sl]], ox1_v.at[sl])
            pltpu.sync_copy(y1c.at[o200_v.at[sl]], oy1_v.at[sl])
            pltpu.sync_copy(x2c.at[o200_v.at[sl]], ox2_v.at[sl])
            pltpu.sync_copy(y2c.at[o200_v.at[sl]], oy2_v.at[sl])
            pltpu.sync_copy(scc.at[o200_v.at[sl]], osc_v.at[sl])
            pltpu.sync_copy(clsc.at[o200_v.at[sl]], ocli_v.at[sl])
        for g in range(OUTP // 16):
            sl = pl.ds(g * 16, 16)
            rv = g * 16 + iota
            osc_v[sl] = jnp.where(rv < ktot, osc_v[sl], -1.0)
            ocl_v[sl] = ocli_v[sl].astype(jnp.float32)
        pltpu.sync_copy(ox1_v, ox1)
        pltpu.sync_copy(oy1_v, oy1)
        pltpu.sync_copy(ox2_v, ox2)
        pltpu.sync_copy(oy2_v, oy2)
        pltpu.sync_copy(osc_v, osc)
        pltpu.sync_copy(ocl_v, ocl)


@jax.jit
def kernel(boxes, scores, classes):
    order = jnp.argsort(-scores).astype(jnp.int32)
    orderp1 = jnp.concatenate(
        [order, jnp.zeros((NP - N,), jnp.int32)])
    x1c = boxes[:, 0]
    y1c = boxes[:, 1]
    x2c = boxes[:, 2]
    y2c = boxes[:, 3]
    clsc = classes.astype(jnp.int32)

    f32 = jnp.float32
    i32 = jnp.int32
    out_type = [jax.ShapeDtypeStruct((OUTP,), f32) for _ in range(6)]
    scratch = [
        pltpu.VMEM((CHW,), i32),
        pltpu.VMEM((CHW,), i32),
        pltpu.VMEM((CHW,), i32),
        pltpu.VMEM((CAP,), i32),
        pltpu.VMEM((CAP,), i32),
        pltpu.VMEM((CAPP,), f32),
        pltpu.VMEM((CAPP,), f32),
        pltpu.VMEM((CAPP,), f32),
        pltpu.VMEM((CAPP,), f32),
        pltpu.VMEM((CAPP,), f32),
        pltpu.VMEM((CAPP,), i32),
        pltpu.VMEM((CAP,), i32),
        pltpu.VMEM((16,), i32),
        pltpu.VMEM((16 * NUM_RES,), i32),
        pltpu.VMEM((CAP,), i32),
        pltpu.VMEM((CAP,), i32),
        pltpu.VMEM((NP,), i32),
        pltpu.VMEM((OUTP,), i32),
        pltpu.VMEM((OUTP,), i32),
        pltpu.VMEM((OUTP,), f32),
        pltpu.VMEM((OUTP,), f32),
        pltpu.VMEM((OUTP,), f32),
        pltpu.VMEM((OUTP,), f32),
        pltpu.VMEM((OUTP,), f32),
        pltpu.VMEM((OUTP,), f32),
        pltpu.VMEM((OUTP,), i32),
        pltpu.VMEM_SHARED((NP,), i32),
        pltpu.VMEM_SHARED((NUM_RES, CAP), i32),
        pltpu.VMEM_SHARED((NUM_RES, CAP), i32),
        pltpu.VMEM_SHARED((16 * NUM_RES,), i32),
    ]
    mesh = plsc.VectorSubcoreMesh(
        core_axis_name="c", subcore_axis_name="s", num_cores=1,
        num_subcores=16)
    run = pl.kernel(
        _nms_body, out_type=out_type, mesh=mesh, scratch_types=scratch,
        compiler_params=pltpu.CompilerParams(needs_layout_passes=False))
    ox1, oy1, ox2, oy2, osc, ocl = run(
        orderp1, x1c, y1c, x2c, y2c, scores, clsc)
    ob = jnp.stack([ox1[:TOPK], oy1[:TOPK], ox2[:TOPK], oy2[:TOPK]], axis=1)
    return jnp.concatenate(
        [ob, osc[:TOPK, None], ocl[:TOPK, None]], axis=1)


# speedup vs baseline: 358.0318x; 1.2966x over previous
import jax
import jax.numpy as jnp
from jax import lax
from jax.experimental import pallas as pl
from jax.experimental.pallas import tpu as pltpu
from jax.experimental.pallas import tpu_sc as plsc

N = 20000
NRES = 32
NSEG = 3
CAPC = 512
CAP = NSEG * CAPC
NP = 20480
CHW = 1280
CAPP = CAP + 16
TOPK = 200
OUTP = 256
IOU_THR = 0.45
GRPS = CAP // 16
NGRP = N // 16


def _iota16():
    return lax.iota(jnp.int32, 16)


def _nms_stage1(orderp1, x1c, y1c, x2c, y2c, clsc,
                pos_o, sup_o, cnt_o,
                och_v, cg_v, cb_v, pos_v, orig_v,
                x1_v, y1_v, x2_v, y2_v, ar_v, keep_v, sup_v,
                cnt16_v, sh_c):
    s = lax.axis_index("s")
    c = lax.axis_index("c")
    w = c * 16 + s
    iota = _iota16()

    pltpu.sync_copy(orderp1.at[pl.ds(s * CHW, CHW)], och_v)
    for j in range(CHW // 128):
        sl = pl.ds(j * 128, 128)
        pltpu.sync_copy(clsc.at[och_v.at[sl]], cg_v.at[sl])
    pltpu.sync_copy(cg_v, sh_c.at[pl.ds(s * CHW, CHW)])
    plsc.subcore_barrier()

    def init_b(g, _):
        off = g * 16
        keep_v[pl.ds(off, 16)] = jnp.ones((16,), jnp.int32)
        pos_v[pl.ds(off, 16)] = jnp.zeros((16,), jnp.int32)
        return 0
    lax.fori_loop(0, GRPS, init_b, 0)

    def filt_chunk(kc, cnts):
        pltpu.sync_copy(sh_c.at[pl.ds(kc * CHW, CHW)], cb_v)

        def filt_g(g, cnts):
            cvec = cb_v[pl.ds(g * 16, 16)]
            pvec = kc * CHW + g * 16 + iota
            pin = pvec < N
            new = []
            for k in range(NSEG):
                ck = cnts[k]
                m = (cvec == (w + NRES * k)) & pin
                mi = m.astype(jnp.int32)
                q = k * CAPC + ck + plsc.cumsum(mi) - mi
                m = m & (q < (k + 1) * CAPC)
                qc = jnp.where(m, q, 0)
                plsc.store_scatter(pos_v, [qc], pvec, mask=m)
                new.append(ck + jnp.sum(mi))
            return tuple(new)
        return lax.fori_loop(0, CHW // 16, filt_g, cnts)
    cnts = lax.fori_loop(0, NP // CHW, filt_chunk,
                         tuple(jnp.int32(0) for _ in range(NSEG)))
    cnts = tuple(jnp.minimum(ck, CAPC) for ck in cnts)
    cntv = jnp.zeros((16,), jnp.int32)
    for k in range(NSEG):
        cntv = jnp.where(iota == k, cnts[k], cntv)
    cnt16_v[...] = cntv

    for k in range(NSEG):
        nch = (cnts[k] + 127) // 128

        def gath(j, _, k=k):
            sl = pl.ds(k * CAPC + j * 128, 128)
            pltpu.sync_copy(orderp1.at[pos_v.at[sl]], orig_v.at[sl])
            pltpu.sync_copy(x1c.at[orig_v.at[sl]], x1_v.at[sl])
            pltpu.sync_copy(y1c.at[orig_v.at[sl]], y1_v.at[sl])
            pltpu.sync_copy(x2c.at[orig_v.at[sl]], x2_v.at[sl])
            pltpu.sync_copy(y2c.at[orig_v.at[sl]], y2_v.at[sl])
            return 0
        lax.fori_loop(0, nch, gath, 0)

        def area_b(g, _, k=k):
            sl = pl.ds(k * CAPC + g * 16, 16)
            wd = jnp.maximum(x2_v[sl] - x1_v[sl], 0.0)
            ht = jnp.maximum(y2_v[sl] - y1_v[sl], 0.0)
            ar_v[sl] = wd * ht
            return 0
        lax.fori_loop(0, (cnts[k] + 15) // 16, area_b, 0)

    for k in range(NSEG):
        base = k * CAPC
        end = base + cnts[k]
        ngrp = (end + 15) // 16

        def nms_i(i, _, end=end, ngrp=ngrp):
            sli = pl.ds(i, 16)
            ki = keep_v[sli][0]

            @pl.when(ki != 0)
            def _():
                xi = jnp.full((16,), x1_v[sli][0], jnp.float32)
                yi = jnp.full((16,), y1_v[sli][0], jnp.float32)
                mXi = jnp.full((16,), x2_v[sli][0], jnp.float32)
                mYi = jnp.full((16,), y2_v[sli][0], jnp.float32)
                ai = jnp.full((16,), ar_v[sli][0], jnp.float32)
                iv = jnp.full((16,), i, jnp.int32)
                ev = jnp.full((16,), end, jnp.int32)

                def nms_g(g, _):
                    off = g * 16
                    sl = pl.ds(off, 16)
                    jv = off + iota
                    xx1 = jnp.maximum(xi, x1_v[sl])
                    yy1 = jnp.maximum(yi, y1_v[sl])
                    xx2 = jnp.minimum(mXi, x2_v[sl])
                    yy2 = jnp.minimum(mYi, y2_v[sl])
                    inter = (jnp.maximum(xx2 - xx1, 0.0)
                             * jnp.maximum(yy2 - yy1, 0.0))
                    iou = inter / (ai + ar_v[sl] - inter + 1e-9)
                    sup = (iou > IOU_THR) & (jv > iv) & (jv < ev)
                    keep_v[sl] = jnp.where(sup, 0, keep_v[sl])
                    return 0
                lax.fori_loop(i // 16, ngrp, nms_g, 0)
            return 0
        lax.fori_loop(base, end, nms_i, 0)

    def sup_b(g, _):
        sl = pl.ds(g * 16, 16)
        jv = g * 16 + iota
        endl = plsc.load_gather(cnt16_v, [jv >> 9])
        sup_v[sl] = ((keep_v[sl] == 0)
                     & ((jv & (CAPC - 1)) < endl)).astype(jnp.int32)
        return 0
    lax.fori_loop(0, GRPS, sup_b, 0)
    pltpu.sync_copy(pos_v, pos_o.at[pl.ds(w * CAP, CAP)])
    pltpu.sync_copy(sup_v, sup_o.at[pl.ds(w * CAP, CAP)])
    pltpu.sync_copy(cnt16_v, cnt_o.at[pl.ds(w * 16, 16)])


def _nms_stage2(orderp1, x1c, y1c, x2c, y2c, scc, clsc,
                pos_i, sup_i, cnt_i,
                ox1, oy1, ox2, oy2, osc, ocl,
                cntall_v, tpos_v, tsup_v, keepg_v,
                r2d_v, o200_v, ox1_v, oy1_v, ox2_v, oy2_v, osc_v,
                ocl_v, ocli_v):
    s = lax.axis_index("s")
    iota = _iota16()

    @pl.when(s == 0)
    def _():
        pltpu.sync_copy(cnt_i, cntall_v)

        def merge_t(t, _):
            pltpu.sync_copy(pos_i.at[pl.ds(t * CAP, CAP)], tpos_v)
            pltpu.sync_copy(sup_i.at[pl.ds(t * CAP, CAP)], tsup_v)
            cvt = cntall_v[pl.ds(t * 16, 16)]
            for k in range(NSEG):
                ct = cvt[k]

                def merge_g(g, _, k=k, ct=ct):
                    off = k * CAPC + g * 16
                    jv = g * 16 + iota
                    pvec = tpos_v[pl.ds(off, 16)]
                    s16 = tsup_v[pl.ds(off, 16)]
                    m = jv < ct
                    pc = jnp.where(m, pvec, 0)
                    plsc.store_scatter(keepg_v, [pc], s16, mask=m)
                    return 0
                lax.fori_loop(0, (ct + 15) // 16, merge_g, 0)
            return 0
        lax.fori_loop(0, NRES, merge_t, 0)

        def kt_b(g, acc):
            sup = keepg_v[pl.ds(g * 16, 16)]
            return acc + jnp.sum(1 - sup)
        ktot = lax.fori_loop(0, NGRP, kt_b, jnp.int32(0))

        for kk in range(OUTP // 16):
            r2d_v[pl.ds(kk * 16, 16)] = jnp.zeros((16,), jnp.int32)

        def part_b(g, cc):
            nk, ns = cc
            sup = keepg_v[pl.ds(g * 16, 16)]
            kii = (sup == 0).astype(jnp.int32)
            sii = 1 - kii
            ck = plsc.cumsum(kii) - kii
            cs = plsc.cumsum(sii) - sii
            pvec = g * 16 + iota
            po = jnp.where(kii == 1, nk + ck, ktot + ns + cs)
            m = po < TOPK
            pc = jnp.where(m, po, 0)
            plsc.store_scatter(r2d_v, [pc], pvec, mask=m)
            return nk + jnp.sum(kii), ns + jnp.sum(sii)
        lax.fori_loop(0, NGRP, part_b, (jnp.int32(0), jnp.int32(0)))

        for j in range(2):
            sl = pl.ds(j * 128, 128)
            pltpu.sync_copy(orderp1.at[r2d_v.at[sl]], o200_v.at[sl])
            pltpu.sync_copy(x1c.at[o200_v.at[sl]], ox1_v.at[sl])
            pltpu.sync_copy(y1c.at[o200_v.at[sl]], oy1_v.at[sl])
            pltpu.sync_copy(x2c.at[o200_v.at[sl]], ox2_v.at[sl])
            pltpu.sync_copy(y2c.at[o200_v.at[sl]], oy2_v.at[sl])
            pltpu.sync_copy(scc.at[o200_v.at[sl]], osc_v.at[sl])
            pltpu.sync_copy(clsc.at[o200_v.at[sl]], ocli_v.at[sl])
        for g in range(OUTP // 16):
            sl = pl.ds(g * 16, 16)
            rv = g * 16 + iota
            osc_v[sl] = jnp.where(rv < ktot, osc_v[sl], -1.0)
            ocl_v[sl] = ocli_v[sl].astype(jnp.float32)
        pltpu.sync_copy(ox1_v, ox1)
        pltpu.sync_copy(oy1_v, oy1)
        pltpu.sync_copy(ox2_v, ox2)
        pltpu.sync_copy(oy2_v, oy2)
        pltpu.sync_copy(osc_v, osc)
        pltpu.sync_copy(ocl_v, ocl)


@jax.jit
def kernel(boxes, scores, classes):
    order = jnp.argsort(-scores).astype(jnp.int32)
    orderp1 = jnp.concatenate(
        [order, jnp.zeros((NP - N,), jnp.int32)])
    x1c = boxes[:, 0]
    y1c = boxes[:, 1]
    x2c = boxes[:, 2]
    y2c = boxes[:, 3]
    clsc = classes.astype(jnp.int32)

    f32 = jnp.float32
    i32 = jnp.int32

    out1 = [jax.ShapeDtypeStruct((NRES * CAP,), i32),
            jax.ShapeDtypeStruct((NRES * CAP,), i32),
            jax.ShapeDtypeStruct((NRES * 16,), i32)]
    scratch1 = [
        pltpu.VMEM((CHW,), i32),
        pltpu.VMEM((CHW,), i32),
        pltpu.VMEM((CHW,), i32),
        pltpu.VMEM((CAP,), i32),
        pltpu.VMEM((CAP,), i32),
        pltpu.VMEM((CAPP,), f32),
        pltpu.VMEM((CAPP,), f32),
        pltpu.VMEM((CAPP,), f32),
        pltpu.VMEM((CAPP,), f32),
        pltpu.VMEM((CAPP,), f32),
        pltpu.VMEM((CAPP,), i32),
        pltpu.VMEM((CAP,), i32),
        pltpu.VMEM((16,), i32),
        pltpu.VMEM_SHARED((NP,), i32),
    ]
    mesh1 = plsc.VectorSubcoreMesh(
        core_axis_name="c", subcore_axis_name="s", num_cores=2,
        num_subcores=16)
    run1 = pl.kernel(
        _nms_stage1, out_type=out1, mesh=mesh1, scratch_types=scratch1,
        compiler_params=pltpu.CompilerParams(needs_layout_passes=False))
    pos_h, sup_h, cnt_h = run1(orderp1, x1c, y1c, x2c, y2c, clsc)

    out2 = [jax.ShapeDtypeStruct((OUTP,), f32) for _ in range(6)]
    scratch2 = [
        pltpu.VMEM((NRES * 16,), i32),
        pltpu.VMEM((CAP,), i32),
        pltpu.VMEM((CAP,), i32),
        pltpu.VMEM((NP,), i32),
        pltpu.VMEM((OUTP,), i32),
        pltpu.VMEM((OUTP,), i32),
        pltpu.VMEM((OUTP,), f32),
        pltpu.VMEM((OUTP,), f32),
        pltpu.VMEM((OUTP,), f32),
        pltpu.VMEM((OUTP,), f32),
        pltpu.VMEM((OUTP,), f32),
        pltpu.VMEM((OUTP,), f32),
        pltpu.VMEM((OUTP,), i32),
    ]
    mesh2 = plsc.VectorSubcoreMesh(
        core_axis_name="c", subcore_axis_name="s", num_cores=1,
        num_subcores=16)
    run2 = pl.kernel(
        _nms_stage2, out_type=out2, mesh=mesh2, scratch_types=scratch2,
        compiler_params=pltpu.CompilerParams(needs_layout_passes=False))
    ox1, oy1, ox2, oy2, osc, ocl = run2(
        orderp1, x1c, y1c, x2c, y2c, scores, clsc, pos_h, sup_h, cnt_h)
    ob = jnp.stack([ox1[:TOPK], oy1[:TOPK], ox2[:TOPK], oy2[:TOPK]], axis=1)
    return jnp.concatenate(
        [ob, osc[:TOPK, None], ocl[:TOPK, None]], axis=1)


# speedup vs baseline: 439.9851x; 1.2289x over previous
import jax
import jax.numpy as jnp
from jax import lax
from jax.experimental import pallas as pl
from jax.experimental.pallas import tpu as pltpu
from jax.experimental.pallas import tpu_sc as plsc

N = 20000
NRES = 32
NSEG = 3
CAPC = 512
CAP = NSEG * CAPC
NP = 20480
CHW = 1280
CAPP = CAP + 16
TOPK = 200
OUTP = 256
IOU_THR = 0.45
GRPS = CAP // 16
NGRP = N // 16


def _iota16():
    return lax.iota(jnp.int32, 16)


def _nms_stage1(orderp1, x1c, y1c, x2c, y2c, clsc,
                pos_o, sup_o, cnt_o,
                och_v, cg_v, cb_v, pos_v, orig_v,
                x1_v, y1_v, x2_v, y2_v, ar_v, keep_v, sup_v,
                cnt16_v, sem, sh_c):
    s = lax.axis_index("s")
    c = lax.axis_index("c")
    w = c * 16 + s
    iota = _iota16()

    pltpu.sync_copy(orderp1.at[pl.ds(s * CHW, CHW)], och_v)
    cps = []
    for j in range(CHW // 128):
        sl = pl.ds(j * 128, 128)
        cps.append(pltpu.async_copy(clsc.at[och_v.at[sl]], cg_v.at[sl], sem))
    for cp in cps:
        cp.wait()
    pltpu.sync_copy(cg_v, sh_c.at[pl.ds(s * CHW, CHW)])
    plsc.subcore_barrier()

    def init_b(g, _):
        off = g * 16
        keep_v[pl.ds(off, 16)] = jnp.ones((16,), jnp.int32)
        pos_v[pl.ds(off, 16)] = jnp.zeros((16,), jnp.int32)
        return 0
    lax.fori_loop(0, GRPS, init_b, 0)

    def filt_chunk(kc, cnts):
        pltpu.sync_copy(sh_c.at[pl.ds(kc * CHW, CHW)], cb_v)

        def filt_g(g, cnts):
            cvec = cb_v[pl.ds(g * 16, 16)]
            pvec = kc * CHW + g * 16 + iota
            pin = pvec < N
            new = []
            for k in range(NSEG):
                ck = cnts[k]
                m = (cvec == (w + NRES * k)) & pin
                mi = m.astype(jnp.int32)
                q = k * CAPC + ck + plsc.cumsum(mi) - mi
                m = m & (q < (k + 1) * CAPC)
                qc = jnp.where(m, q, 0)
                plsc.store_scatter(pos_v, [qc], pvec, mask=m)
                new.append(ck + jnp.sum(mi))
            return tuple(new)
        return lax.fori_loop(0, CHW // 16, filt_g, cnts)
    cnts = lax.fori_loop(0, NP // CHW, filt_chunk,
                         tuple(jnp.int32(0) for _ in range(NSEG)))
    cnts = tuple(jnp.minimum(ck, CAPC) for ck in cnts)
    cntv = jnp.zeros((16,), jnp.int32)
    for k in range(NSEG):
        cntv = jnp.where(iota == k, cnts[k], cntv)
    cnt16_v[...] = cntv

    for k in range(NSEG):
        nch = (cnts[k] + 127) // 128

        def gath(j, _, k=k):
            sl = pl.ds(k * CAPC + j * 128, 128)
            pltpu.sync_copy(orderp1.at[pos_v.at[sl]], orig_v.at[sl])
            c1 = pltpu.async_copy(x1c.at[orig_v.at[sl]], x1_v.at[sl], sem)
            c2 = pltpu.async_copy(y1c.at[orig_v.at[sl]], y1_v.at[sl], sem)
            c3 = pltpu.async_copy(x2c.at[orig_v.at[sl]], x2_v.at[sl], sem)
            c4 = pltpu.async_copy(y2c.at[orig_v.at[sl]], y2_v.at[sl], sem)
            c1.wait(); c2.wait(); c3.wait(); c4.wait()
            return 0
        lax.fori_loop(0, nch, gath, 0)

        def area_b(g, _, k=k):
            sl = pl.ds(k * CAPC + g * 16, 16)
            wd = jnp.maximum(x2_v[sl] - x1_v[sl], 0.0)
            ht = jnp.maximum(y2_v[sl] - y1_v[sl], 0.0)
            ar_v[sl] = wd * ht
            return 0
        lax.fori_loop(0, (cnts[k] + 15) // 16, area_b, 0)

    for k in range(NSEG):
        base = k * CAPC
        end = base + cnts[k]
        ngrp = (end + 15) // 16

        def nms_i(i, _, end=end, ngrp=ngrp):
            sli = pl.ds(i, 16)
            ki = keep_v[sli][0]

            @pl.when(ki != 0)
            def _():
                xi = jnp.full((16,), x1_v[sli][0], jnp.float32)
                yi = jnp.full((16,), y1_v[sli][0], jnp.float32)
                mXi = jnp.full((16,), x2_v[sli][0], jnp.float32)
                mYi = jnp.full((16,), y2_v[sli][0], jnp.float32)
                ai = jnp.full((16,), ar_v[sli][0], jnp.float32)
                iv = jnp.full((16,), i, jnp.int32)
                ev = jnp.full((16,), end, jnp.int32)

                def nms_g(g, _):
                    off = g * 16
                    sl = pl.ds(off, 16)
                    jv = off + iota
                    xx1 = jnp.maximum(xi, x1_v[sl])
                    yy1 = jnp.maximum(yi, y1_v[sl])
                    xx2 = jnp.minimum(mXi, x2_v[sl])
                    yy2 = jnp.minimum(mYi, y2_v[sl])
                    inter = (jnp.maximum(xx2 - xx1, 0.0)
                             * jnp.maximum(yy2 - yy1, 0.0))
                    iou = inter / (ai + ar_v[sl] - inter + 1e-9)
                    sup = (iou > IOU_THR) & (jv > iv) & (jv < ev)
                    keep_v[sl] = jnp.where(sup, 0, keep_v[sl])
                    return 0
                lax.fori_loop(i // 16, ngrp, nms_g, 0)
            return 0
        lax.fori_loop(base, end, nms_i, 0)

    def sup_b(g, _):
        sl = pl.ds(g * 16, 16)
        jv = g * 16 + iota
        endl = plsc.load_gather(cnt16_v, [jv >> 9])
        sup_v[sl] = ((keep_v[sl] == 0)
                     & ((jv & (CAPC - 1)) < endl)).astype(jnp.int32)
        return 0
    lax.fori_loop(0, GRPS, sup_b, 0)
    pltpu.sync_copy(pos_v, pos_o.at[pl.ds(w * CAP, CAP)])
    pltpu.sync_copy(sup_v, sup_o.at[pl.ds(w * CAP, CAP)])
    pltpu.sync_copy(cnt16_v, cnt_o.at[pl.ds(w * 16, 16)])


def _nms_stage2(orderp1, x1c, y1c, x2c, y2c, scc, clsc,
                pos_i, sup_i, cnt_i,
                ox1, oy1, ox2, oy2, osc, ocl,
                cntall_v, tpos_v, tsup_v, keepg_v, idxb_v, zb_v, s16b_v,
                r2d_v, o200_v, ox1_v, oy1_v, ox2_v, oy2_v, osc_v,
                ocl_v, ocli_v, sh_keep, sh_sums):
    s = lax.axis_index("s")
    iota = _iota16()

    def z_b(g, _):
        zb_v[pl.ds(g * 16, 16)] = jnp.zeros((16,), jnp.int32)
        return 0
    lax.fori_loop(0, (NP // 16) // 16, z_b, 0)
    pltpu.sync_copy(zb_v, sh_keep.at[pl.ds(s * (NP // 16), NP // 16)])
    pltpu.sync_copy(cnt_i, cntall_v)
    plsc.subcore_barrier()

    ssum = jnp.int32(0)
    for t_off in (0, 16):
        t = s + t_off
        pltpu.sync_copy(pos_i.at[pl.ds(t * CAP, CAP)], tpos_v)
        pltpu.sync_copy(sup_i.at[pl.ds(t * CAP, CAP)], tsup_v)

        def sum_b(g, acc):
            return acc + jnp.sum(tsup_v[pl.ds(g * 16, 16)])
        ssum = ssum + lax.fori_loop(0, GRPS, sum_b, jnp.int32(0))
        cvt = cntall_v[pl.ds(t * 16, 16)]
        for k in range(NSEG):
            ct = cvt[k]

            def add_ch(j, _, k=k, ct=ct, t=t):
                base = k * CAPC + j * 128
                for u in range(8):
                    lanej = j * 128 + u * 16 + iota
                    pv = tpos_v[pl.ds(base + u * 16, 16)]
                    idxb_v[pl.ds(u * 16, 16)] = jnp.where(
                        lanej < ct, pv, NP + t * CAP + k * CAPC + lanej)
                pltpu.sync_copy(tsup_v.at[pl.ds(base, 128)],
                                sh_keep.at[idxb_v], add=True)
                return 0
            lax.fori_loop(0, (ct + 127) // 128, add_ch, 0)
    s16b_v[...] = jnp.full((16,), ssum, jnp.int32)
    pltpu.sync_copy(s16b_v, sh_sums.at[pl.ds(s * 16, 16)])
    plsc.subcore_barrier()

    @pl.when(s == 0)
    def _():
        pltpu.sync_copy(sh_keep.at[pl.ds(0, NP)], keepg_v)
        pltpu.sync_copy(sh_sums, zb_v.at[pl.ds(0, 16 * 16)])

        def st_b(t, acc):
            return acc + zb_v[pl.ds(t * 16, 16)][0]
        tot_sup = lax.fori_loop(0, 16, st_b, jnp.int32(0))
        ktot = N - tot_sup

        for kk in range(OUTP // 16):
            r2d_v[pl.ds(kk * 16, 16)] = jnp.zeros((16,), jnp.int32)

        def part_b(g, cc):
            nk, ns = cc
            sup = keepg_v[pl.ds(g * 16, 16)]
            kii = (sup == 0).astype(jnp.int32)
            sii = 1 - kii
            ck = plsc.cumsum(kii) - kii
            cs = plsc.cumsum(sii) - sii
            pvec = g * 16 + iota
            po = jnp.where(kii == 1, nk + ck, ktot + ns + cs)
            m = po < TOPK
            pc = jnp.where(m, po, 0)
            plsc.store_scatter(r2d_v, [pc], pvec, mask=m)
            return nk + jnp.sum(kii), ns + jnp.sum(sii)
        lax.fori_loop(0, NGRP, part_b, (jnp.int32(0), jnp.int32(0)))

        for j in range(2):
            sl = pl.ds(j * 128, 128)
            pltpu.sync_copy(orderp1.at[r2d_v.at[sl]], o200_v.at[sl])
            pltpu.sync_copy(x1c.at[o200_v.at[sl]], ox1_v.at[sl])
            pltpu.sync_copy(y1c.at[o200_v.at[sl]], oy1_v.at[sl])
            pltpu.sync_copy(x2c.at[o200_v.at[sl]], ox2_v.at[sl])
            pltpu.sync_copy(y2c.at[o200_v.at[sl]], oy2_v.at[sl])
            pltpu.sync_copy(scc.at[o200_v.at[sl]], osc_v.at[sl])
            pltpu.sync_copy(clsc.at[o200_v.at[sl]], ocli_v.at[sl])
        for g in range(OUTP // 16):
            sl = pl.ds(g * 16, 16)
            rv = g * 16 + iota
            osc_v[sl] = jnp.where(rv < ktot, osc_v[sl], -1.0)
            ocl_v[sl] = ocli_v[sl].astype(jnp.float32)
        pltpu.sync_copy(ox1_v, ox1)
        pltpu.sync_copy(oy1_v, oy1)
        pltpu.sync_copy(ox2_v, ox2)
        pltpu.sync_copy(oy2_v, oy2)
        pltpu.sync_copy(osc_v, osc)
        pltpu.sync_copy(ocl_v, ocl)


@jax.jit
def kernel(boxes, scores, classes):
    order = jnp.argsort(-scores).astype(jnp.int32)
    orderp1 = jnp.concatenate(
        [order, jnp.zeros((NP - N,), jnp.int32)])
    x1c = boxes[:, 0]
    y1c = boxes[:, 1]
    x2c = boxes[:, 2]
    y2c = boxes[:, 3]
    clsc = classes.astype(jnp.int32)

    f32 = jnp.float32
    i32 = jnp.int32

    out1 = [jax.ShapeDtypeStruct((NRES * CAP,), i32),
            jax.ShapeDtypeStruct((NRES * CAP,), i32),
            jax.ShapeDtypeStruct((NRES * 16,), i32)]
    scratch1 = [
        pltpu.VMEM((CHW,), i32),
        pltpu.VMEM((CHW,), i32),
        pltpu.VMEM((CHW,), i32),
        pltpu.VMEM((CAP,), i32),
        pltpu.VMEM((CAP,), i32),
        pltpu.VMEM((CAPP,), f32),
        pltpu.VMEM((CAPP,), f32),
        pltpu.VMEM((CAPP,), f32),
        pltpu.VMEM((CAPP,), f32),
        pltpu.VMEM((CAPP,), f32),
        pltpu.VMEM((CAPP,), i32),
        pltpu.VMEM((CAP,), i32),
        pltpu.VMEM((16,), i32),
        pltpu.SemaphoreType.DMA,
        pltpu.VMEM_SHARED((NP,), i32),
    ]
    mesh1 = plsc.VectorSubcoreMesh(
        core_axis_name="c", subcore_axis_name="s", num_cores=2,
        num_subcores=16)
    run1 = pl.kernel(
        _nms_stage1, out_type=out1, mesh=mesh1, scratch_types=scratch1,
        compiler_params=pltpu.CompilerParams(needs_layout_passes=False))
    pos_h, sup_h, cnt_h = run1(orderp1, x1c, y1c, x2c, y2c, clsc)

    out2 = [jax.ShapeDtypeStruct((OUTP,), f32) for _ in range(6)]
    scratch2 = [
        pltpu.VMEM((NRES * 16,), i32),
        pltpu.VMEM((CAP,), i32),
        pltpu.VMEM((CAP,), i32),
        pltpu.VMEM((NP,), i32),
        pltpu.VMEM((128,), i32),
        pltpu.VMEM((NP // 16,), i32),
        pltpu.VMEM((16,), i32),
        pltpu.VMEM((OUTP,), i32),
        pltpu.VMEM((OUTP,), i32),
        pltpu.VMEM((OUTP,), f32),
        pltpu.VMEM((OUTP,), f32),
        pltpu.VMEM((OUTP,), f32),
        pltpu.VMEM((OUTP,), f32),
        pltpu.VMEM((OUTP,), f32),
        pltpu.VMEM((OUTP,), f32),
        pltpu.VMEM((OUTP,), i32),
        pltpu.VMEM_SHARED((NP + NRES * CAP,), i32),
        pltpu.VMEM_SHARED((16 * 16,), i32),
    ]
    mesh2 = plsc.VectorSubcoreMesh(
        core_axis_name="c", subcore_axis_name="s", num_cores=1,
        num_subcores=16)
    run2 = pl.kernel(
        _nms_stage2, out_type=out2, mesh=mesh2, scratch_types=scratch2,
        compiler_params=pltpu.CompilerParams(needs_layout_passes=False))
    ox1, oy1, ox2, oy2, osc, ocl = run2(
        orderp1, x1c, y1c, x2c, y2c, scores, clsc, pos_h, sup_h, cnt_h)
    ob = jnp.stack([ox1[:TOPK], oy1[:TOPK], ox2[:TOPK], oy2[:TOPK]], axis=1)
    return jnp.concatenate(
        [ob, osc[:TOPK, None], ocl[:TOPK, None]], axis=1)


# speedup vs baseline: 444.1741x; 1.0095x over previous
import jax
import jax.numpy as jnp
from jax import lax
from jax.experimental import pallas as pl
from jax.experimental.pallas import tpu as pltpu
from jax.experimental.pallas import tpu_sc as plsc

N = 20000
NRES = 32
NSEG = 3
CAPC = 512
CAP = NSEG * CAPC
NP = 20480
CHW = 1280
CAPP = CAP + 16
TOPK = 200
OUTP = 256
IOU_THR = 0.45
GRPS = CAP // 16
NGRP = N // 16


def _iota16():
    return lax.iota(jnp.int32, 16)


def _nms_stage1(orderp1, x1c, y1c, x2c, y2c, clsc,
                pos_o, sup_o, cnt_o,
                och_v, cg_v, cb_v, pos_v, orig_v,
                x1_v, y1_v, x2_v, y2_v, ar_v, keep_v, sup_v,
                cnt16_v, sem, sh_c):
    s = lax.axis_index("s")
    c = lax.axis_index("c")
    w = c * 16 + s
    iota = _iota16()

    pltpu.sync_copy(orderp1.at[pl.ds(s * CHW, CHW)], och_v)
    cps = []
    for j in range(CHW // 128):
        sl = pl.ds(j * 128, 128)
        cps.append(pltpu.async_copy(clsc.at[och_v.at[sl]], cg_v.at[sl], sem))
    for cp in cps:
        cp.wait()
    pltpu.sync_copy(cg_v, sh_c.at[pl.ds(s * CHW, CHW)])
    plsc.subcore_barrier()

    def init_b(g, _):
        off = g * 16
        keep_v[pl.ds(off, 16)] = jnp.ones((16,), jnp.int32)
        pos_v[pl.ds(off, 16)] = jnp.zeros((16,), jnp.int32)
        return 0
    lax.fori_loop(0, GRPS, init_b, 0)

    pltpu.sync_copy(sh_c, cb_v)

    def filt_g(g, cnts):
        cvec = cb_v[pl.ds(g * 16, 16)]
        pvec = g * 16 + iota
        pin = pvec < N
        new = []
        for k in range(NSEG):
            ck = cnts[k]
            m = (cvec == (w + NRES * k)) & pin
            mi = m.astype(jnp.int32)
            q = k * CAPC + ck + plsc.cumsum(mi) - mi
            m = m & (q < (k + 1) * CAPC)
            qc = jnp.where(m, q, 0)
            plsc.store_scatter(pos_v, [qc], pvec, mask=m)
            new.append(ck + jnp.sum(mi))
        return tuple(new)
    cnts = lax.fori_loop(0, NP // 16, filt_g,
                         tuple(jnp.int32(0) for _ in range(NSEG)))
    cnts = tuple(jnp.minimum(ck, CAPC) for ck in cnts)
    cntv = jnp.zeros((16,), jnp.int32)
    for k in range(NSEG):
        cntv = jnp.where(iota == k, cnts[k], cntv)
    cnt16_v[...] = cntv

    for k in range(NSEG):
        nch = (cnts[k] + 127) // 128

        def gath(j, _, k=k):
            sl = pl.ds(k * CAPC + j * 128, 128)
            pltpu.sync_copy(orderp1.at[pos_v.at[sl]], orig_v.at[sl])
            c1 = pltpu.async_copy(x1c.at[orig_v.at[sl]], x1_v.at[sl], sem)
            c2 = pltpu.async_copy(y1c.at[orig_v.at[sl]], y1_v.at[sl], sem)
            c3 = pltpu.async_copy(x2c.at[orig_v.at[sl]], x2_v.at[sl], sem)
            c4 = pltpu.async_copy(y2c.at[orig_v.at[sl]], y2_v.at[sl], sem)
            c1.wait(); c2.wait(); c3.wait(); c4.wait()
            return 0
        lax.fori_loop(0, nch, gath, 0)

        def area_b(g, _, k=k):
            sl = pl.ds(k * CAPC + g * 16, 16)
            wd = jnp.maximum(x2_v[sl] - x1_v[sl], 0.0)
            ht = jnp.maximum(y2_v[sl] - y1_v[sl], 0.0)
            ar_v[sl] = wd * ht
            return 0
        lax.fori_loop(0, (cnts[k] + 15) // 16, area_b, 0)

    for k in range(NSEG):
        base = k * CAPC
        end = base + cnts[k]
        ngrp = (end + 15) // 16

        def nms_i(i, _, end=end, ngrp=ngrp):
            sli = pl.ds(i, 16)
            ki = keep_v[sli][0]

            @pl.when(ki != 0)
            def _():
                xi = jnp.full((16,), x1_v[sli][0], jnp.float32)
                yi = jnp.full((16,), y1_v[sli][0], jnp.float32)
                mXi = jnp.full((16,), x2_v[sli][0], jnp.float32)
                mYi = jnp.full((16,), y2_v[sli][0], jnp.float32)
                ai = jnp.full((16,), ar_v[sli][0], jnp.float32)
                iv = jnp.full((16,), i, jnp.int32)
                ev = jnp.full((16,), end, jnp.int32)

                def nms_g(g, _):
                    off = g * 16
                    sl = pl.ds(off, 16)
                    jv = off + iota
                    xx1 = jnp.maximum(xi, x1_v[sl])
                    yy1 = jnp.maximum(yi, y1_v[sl])
                    xx2 = jnp.minimum(mXi, x2_v[sl])
                    yy2 = jnp.minimum(mYi, y2_v[sl])
                    inter = (jnp.maximum(xx2 - xx1, 0.0)
                             * jnp.maximum(yy2 - yy1, 0.0))
                    iou = inter / (ai + ar_v[sl] - inter + 1e-9)
                    sup = (iou > IOU_THR) & (jv > iv) & (jv < ev)
                    keep_v[sl] = jnp.where(sup, 0, keep_v[sl])
                    return 0
                lax.fori_loop(i // 16, ngrp, nms_g, 0)
            return 0
        lax.fori_loop(base, end, nms_i, 0)

    def sup_b(g, _):
        sl = pl.ds(g * 16, 16)
        jv = g * 16 + iota
        endl = plsc.load_gather(cnt16_v, [jv >> 9])
        sup_v[sl] = ((keep_v[sl] == 0)
                     & ((jv & (CAPC - 1)) < endl)).astype(jnp.int32)
        return 0
    lax.fori_loop(0, GRPS, sup_b, 0)
    pltpu.sync_copy(pos_v, pos_o.at[pl.ds(w * CAP, CAP)])
    pltpu.sync_copy(sup_v, sup_o.at[pl.ds(w * CAP, CAP)])
    pltpu.sync_copy(cnt16_v, cnt_o.at[pl.ds(w * 16, 16)])


def _nms_stage2(orderp1, x1c, y1c, x2c, y2c, scc, clsc,
                pos_i, sup_i, cnt_i,
                ox1, oy1, ox2, oy2, osc, ocl,
                cntall_v, tpos_v, tsup_v, keepg_v, idxb_v, zb_v, s16b_v,
                r2d_v, o200_v, ox1_v, oy1_v, ox2_v, oy2_v, osc_v,
                ocl_v, ocli_v, sh_keep, sh_sums):
    s = lax.axis_index("s")
    iota = _iota16()

    def z_b(g, _):
        zb_v[pl.ds(g * 16, 16)] = jnp.zeros((16,), jnp.int32)
        return 0
    lax.fori_loop(0, (NP // 16) // 16, z_b, 0)
    pltpu.sync_copy(zb_v, sh_keep.at[pl.ds(s * (NP // 16), NP // 16)])
    pltpu.sync_copy(cnt_i, cntall_v)
    plsc.subcore_barrier()

    ssum = jnp.int32(0)
    for t_off in (0, 16):
        t = s + t_off
        pltpu.sync_copy(pos_i.at[pl.ds(t * CAP, CAP)], tpos_v)
        pltpu.sync_copy(sup_i.at[pl.ds(t * CAP, CAP)], tsup_v)

        def sum_b(g, acc):
            return acc + jnp.sum(tsup_v[pl.ds(g * 16, 16)])
        ssum = ssum + lax.fori_loop(0, GRPS, sum_b, jnp.int32(0))
        cvt = cntall_v[pl.ds(t * 16, 16)]
        for k in range(NSEG):
            ct = cvt[k]

            def add_ch(j, _, k=k, ct=ct, t=t):
                base = k * CAPC + j * 128
                for u in range(8):
                    lanej = j * 128 + u * 16 + iota
                    pv = tpos_v[pl.ds(base + u * 16, 16)]
                    idxb_v[pl.ds(u * 16, 16)] = jnp.where(
                        lanej < ct, pv, NP + t * CAP + k * CAPC + lanej)
                pltpu.sync_copy(tsup_v.at[pl.ds(base, 128)],
                                sh_keep.at[idxb_v], add=True)
                return 0
            lax.fori_loop(0, (ct + 127) // 128, add_ch, 0)
    s16b_v[...] = jnp.full((16,), ssum, jnp.int32)
    pltpu.sync_copy(s16b_v, sh_sums.at[pl.ds(s * 16, 16)])
    plsc.subcore_barrier()

    @pl.when(s == 0)
    def _():
        pltpu.sync_copy(sh_keep.at[pl.ds(0, NP)], keepg_v)
        pltpu.sync_copy(sh_sums, zb_v.at[pl.ds(0, 16 * 16)])

        def st_b(t, acc):
            return acc + zb_v[pl.ds(t * 16, 16)][0]
        tot_sup = lax.fori_loop(0, 16, st_b, jnp.int32(0))
        ktot = N - tot_sup

        for kk in range(OUTP // 16):
            r2d_v[pl.ds(kk * 16, 16)] = jnp.zeros((16,), jnp.int32)

        def part_b(g, cc):
            nk, ns = cc
            sup = keepg_v[pl.ds(g * 16, 16)]
            kii = (sup == 0).astype(jnp.int32)
            sii = 1 - kii
            ck = plsc.cumsum(kii) - kii
            cs = plsc.cumsum(sii) - sii
            pvec = g * 16 + iota
            po = jnp.where(kii == 1, nk + ck, ktot + ns + cs)
            m = po < TOPK
            pc = jnp.where(m, po, 0)
            plsc.store_scatter(r2d_v, [pc], pvec, mask=m)
            return nk + jnp.sum(kii), ns + jnp.sum(sii)
        lax.fori_loop(0, NGRP, part_b, (jnp.int32(0), jnp.int32(0)))

        for j in range(2):
            sl = pl.ds(j * 128, 128)
            pltpu.sync_copy(orderp1.at[r2d_v.at[sl]], o200_v.at[sl])
            pltpu.sync_copy(x1c.at[o200_v.at[sl]], ox1_v.at[sl])
            pltpu.sync_copy(y1c.at[o200_v.at[sl]], oy1_v.at[sl])
            pltpu.sync_copy(x2c.at[o200_v.at[sl]], ox2_v.at[sl])
            pltpu.sync_copy(y2c.at[o200_v.at[sl]], oy2_v.at[sl])
            pltpu.sync_copy(scc.at[o200_v.at[sl]], osc_v.at[sl])
            pltpu.sync_copy(clsc.at[o200_v.at[sl]], ocli_v.at[sl])
        for g in range(OUTP // 16):
            sl = pl.ds(g * 16, 16)
            rv = g * 16 + iota
            osc_v[sl] = jnp.where(rv < ktot, osc_v[sl], -1.0)
            ocl_v[sl] = ocli_v[sl].astype(jnp.float32)
        pltpu.sync_copy(ox1_v, ox1)
        pltpu.sync_copy(oy1_v, oy1)
        pltpu.sync_copy(ox2_v, ox2)
        pltpu.sync_copy(oy2_v, oy2)
        pltpu.sync_copy(osc_v, osc)
        pltpu.sync_copy(ocl_v, ocl)


@jax.jit
def kernel(boxes, scores, classes):
    order = jnp.argsort(-scores).astype(jnp.int32)
    orderp1 = jnp.concatenate(
        [order, jnp.zeros((NP - N,), jnp.int32)])
    x1c = boxes[:, 0]
    y1c = boxes[:, 1]
    x2c = boxes[:, 2]
    y2c = boxes[:, 3]
    clsc = classes.astype(jnp.int32)

    f32 = jnp.float32
    i32 = jnp.int32

    out1 = [jax.ShapeDtypeStruct((NRES * CAP,), i32),
            jax.ShapeDtypeStruct((NRES * CAP,), i32),
            jax.ShapeDtypeStruct((NRES * 16,), i32)]
    scratch1 = [
        pltpu.VMEM((CHW,), i32),
        pltpu.VMEM((CHW,), i32),
        pltpu.VMEM((NP,), i32),
        pltpu.VMEM((CAP,), i32),
        pltpu.VMEM((CAP,), i32),
        pltpu.VMEM((CAPP,), f32),
        pltpu.VMEM((CAPP,), f32),
        pltpu.VMEM((CAPP,), f32),
        pltpu.VMEM((CAPP,), f32),
        pltpu.VMEM((CAPP,), f32),
        pltpu.VMEM((CAPP,), i32),
        pltpu.VMEM((CAP,), i32),
        pltpu.VMEM((16,), i32),
        pltpu.SemaphoreType.DMA,
        pltpu.VMEM_SHARED((NP,), i32),
    ]
    mesh1 = plsc.VectorSubcoreMesh(
        core_axis_name="c", subcore_axis_name="s", num_cores=2,
        num_subcores=16)
    run1 = pl.kernel(
        _nms_stage1, out_type=out1, mesh=mesh1, scratch_types=scratch1,
        compiler_params=pltpu.CompilerParams(needs_layout_passes=False))
    pos_h, sup_h, cnt_h = run1(orderp1, x1c, y1c, x2c, y2c, clsc)

    out2 = [jax.ShapeDtypeStruct((OUTP,), f32) for _ in range(6)]
    scratch2 = [
        pltpu.VMEM((NRES * 16,), i32),
        pltpu.VMEM((CAP,), i32),
        pltpu.VMEM((CAP,), i32),
        pltpu.VMEM((NP,), i32),
        pltpu.VMEM((128,), i32),
        pltpu.VMEM((NP // 16,), i32),
        pltpu.VMEM((16,), i32),
        pltpu.VMEM((OUTP,), i32),
        pltpu.VMEM((OUTP,), i32),
        pltpu.VMEM((OUTP,), f32),
        pltpu.VMEM((OUTP,), f32),
        pltpu.VMEM((OUTP,), f32),
        pltpu.VMEM((OUTP,), f32),
        pltpu.VMEM((OUTP,), f32),
        pltpu.VMEM((OUTP,), f32),
        pltpu.VMEM((OUTP,), i32),
        pltpu.VMEM_SHARED((NP + NRES * CAP,), i32),
        pltpu.VMEM_SHARED((16 * 16,), i32),
    ]
    mesh2 = plsc.VectorSubcoreMesh(
        core_axis_name="c", subcore_axis_name="s", num_cores=1,
        num_subcores=16)
    run2 = pl.kernel(
        _nms_stage2, out_type=out2, mesh=mesh2, scratch_types=scratch2,
        compiler_params=pltpu.CompilerParams(needs_layout_passes=False))
    ox1, oy1, ox2, oy2, osc, ocl = run2(
        orderp1, x1c, y1c, x2c, y2c, scores, clsc, pos_h, sup_h, cnt_h)
    ob = jnp.stack([ox1[:TOPK], oy1[:TOPK], ox2[:TOPK], oy2[:TOPK]], axis=1)
    return jnp.concatenate(
        [ob, osc[:TOPK, None], ocl[:TOPK, None]], axis=1)
